# baseline (speedup 1.0000x reference)
import functools

import jax
import jax.numpy as jnp
from jax import lax
from jax.experimental import pallas as pl
from jax.experimental.pallas import tpu as pltpu
from jax.experimental.pallas import tpu_sc as plsc

N_NODES = 10000
N_EDGES = 320000
HIDDEN = 128

NUM_CORES = 2
NUM_SUBCORES = 16
NW = NUM_CORES * NUM_SUBCORES
CHUNK = 64
NBUF = 5
NGROUPS = 32
CHUNKS_PER_TILE = NBUF * NGROUPS
E_TILE = CHUNK * CHUNKS_PER_TILE
E_PAD = E_TILE * NW
ROW_CHUNK = 16
N_ROW_CHUNKS = N_NODES // ROW_CHUNK
LANES = 16


def _matmul_body(x_ref, w_ref, o_ref):
    o_ref[...] = lax.dot_general(
        x_ref[...], w_ref[...], (((1,), (0,)), ((), ())),
        preferred_element_type=jnp.float32,
        precision=lax.Precision.HIGHEST,
    )


def _support(x, W):
    return pl.pallas_call(
        _matmul_body,
        grid=(10,),
        in_specs=[
            pl.BlockSpec((N_NODES // 10, HIDDEN), lambda i: (i, 0)),
            pl.BlockSpec((HIDDEN, HIDDEN), lambda i: (0, 0)),
        ],
        out_specs=pl.BlockSpec((N_NODES // 10, HIDDEN), lambda i: (i, 0)),
        out_shape=jax.ShapeDtypeStruct((N_NODES, HIDDEN), jnp.float32),
    )(x, W)


def _finish_body(p_ref, b_ref, o_ref):
    o_ref[...] = p_ref[0] + p_ref[1] + b_ref[...]


def _finish(partials, b2d):
    return pl.pallas_call(
        _finish_body,
        grid=(10,),
        in_specs=[
            pl.BlockSpec((2, N_NODES // 10, HIDDEN), lambda i: (0, i, 0)),
            pl.BlockSpec((1, HIDDEN), lambda i: (0, 0)),
        ],
        out_specs=pl.BlockSpec((N_NODES // 10, HIDDEN), lambda i: (i, 0)),
        out_shape=jax.ShapeDtypeStruct((N_NODES, HIDDEN), jnp.float32),
    )(partials, b2d)


def _aggregate(support, src3, dst3, val3):
    mesh = plsc.VectorSubcoreMesh(core_axis_name="c", subcore_axis_name="s",
                                  num_cores=NUM_CORES, num_subcores=NUM_SUBCORES)

    @functools.partial(
        pl.kernel,
        out_type=jax.ShapeDtypeStruct((NUM_CORES, N_NODES, HIDDEN), jnp.float32),
        mesh=mesh,
        scratch_types=[
            pltpu.VMEM((2, NBUF, CHUNK), jnp.int32),
            pltpu.VMEM((2, NBUF, CHUNK), jnp.int32),
            pltpu.VMEM((2, NBUF, CHUNK), jnp.float32),
            pltpu.VMEM((NBUF, CHUNK, HIDDEN), jnp.float32),
            pltpu.VMEM_SHARED((N_NODES, HIDDEN), jnp.float32),
            [pltpu.SemaphoreType.DMA] * NBUF,
            [pltpu.SemaphoreType.DMA] * NBUF,
            [pltpu.SemaphoreType.DMA] * 2,
        ],
    )
    def agg_kernel(sup_hbm, src_hbm, dst_hbm, val_hbm, out_hbm,
                   src_v, dst_v, val_v, rows, acc, gsem, ssem, isem):
        c = lax.axis_index("c")
        s = lax.axis_index("s")
        wid = c * NUM_SUBCORES + s

        def prefetch_idx(group, slot, sem):
            pltpu.async_copy(src_hbm.at[wid, group], src_v.at[slot], sem)
            pltpu.async_copy(dst_hbm.at[wid, group], dst_v.at[slot], sem)
            pltpu.async_copy(val_hbm.at[wid, group], val_v.at[slot], sem)

        def wait_idx(slot, sem):
            pltpu.make_async_copy(src_hbm.at[wid, 0], src_v.at[slot], sem).wait()
            pltpu.make_async_copy(dst_hbm.at[wid, 0], dst_v.at[slot], sem).wait()
            pltpu.make_async_copy(val_hbm.at[wid, 0], val_v.at[slot], sem).wait()

        prefetch_idx(0, 0, isem[0])
        prefetch_idx(1, 1, isem[1])
        wait_idx(0, isem[0])

        @pl.loop(0, ROW_CHUNK)
        def _zero_rows(r):
            for h in range(0, HIDDEN, LANES):
                rows[0, r, pl.ds(h, LANES)] = jnp.zeros((LANES,), jnp.float32)

        @pl.loop(0, N_ROW_CHUNKS // NUM_SUBCORES + 1)
        def _zero_acc(k):
            cidx = s + k * NUM_SUBCORES
            @pl.when(cidx < N_ROW_CHUNKS)
            def _():
                pltpu.sync_copy(rows.at[0, pl.ds(0, ROW_CHUNK)],
                                acc.at[pl.ds(cidx * ROW_CHUNK, ROW_CHUNK)])

        plsc.subcore_barrier()

        def process_group(gg, slot, other_ready):
            for b in range(NBUF):
                pltpu.make_async_copy(sup_hbm.at[src_v.at[slot, b]],
                                      rows.at[b], gsem[b]).wait()

                @pl.loop(0, CHUNK, step=LANES)
                def _scale(g0):
                    vv = val_v[slot, b, pl.ds(g0, LANES)]
                    for e in range(LANES):
                        bc = jnp.broadcast_to(vv[e], (LANES,))
                        for h in range(0, HIDDEN, LANES):
                            rows[b, g0 + e, pl.ds(h, LANES)] = (
                                rows[b, g0 + e, pl.ds(h, LANES)] * bc)

                pltpu.async_copy(rows.at[b], acc.at[dst_v.at[slot, b]],
                                 ssem[b], add=True)

            @pl.when(gg + 1 < NGROUPS)
            def _refill():
                if other_ready is not None:
                    wait_idx(1 - slot, other_ready)
                for b in range(NBUF):
                    pltpu.make_async_copy(rows.at[b],
                                          acc.at[dst_v.at[slot, b]],
                                          ssem[b]).wait()
                    pltpu.async_copy(sup_hbm.at[src_v.at[1 - slot, b]],
                                     rows.at[b], gsem[b])

            @pl.when(gg + 2 < NGROUPS)
            def _prefetch():
                prefetch_idx(gg + 2, slot, isem[slot])

        for b in range(NBUF):
            pltpu.async_copy(sup_hbm.at[src_v.at[0, b]], rows.at[b], gsem[b])

        @pl.loop(0, NGROUPS, step=2)
        def _pair(g):
            process_group(g, 0, isem[1])
            process_group(g + 1, 1, isem[0])

        for b in range(NBUF):
            pltpu.make_async_copy(rows.at[b], acc.at[dst_v.at[1, b]],
                                  ssem[b]).wait()

        plsc.subcore_barrier()

        @pl.loop(0, N_ROW_CHUNKS // NUM_SUBCORES + 1)
        def _writeback(k):
            cidx = s + k * NUM_SUBCORES
            @pl.when(cidx < N_ROW_CHUNKS)
            def _():
                base = cidx * ROW_CHUNK
                pltpu.sync_copy(acc.at[pl.ds(base, ROW_CHUNK)],
                                out_hbm.at[c, pl.ds(base, ROW_CHUNK)])

    return agg_kernel(support, src3, dst3, val3)


def kernel(x, edge_index, edge_values, W, b):
    src = edge_index[1].astype(jnp.int32)
    dst = edge_index[0].astype(jnp.int32)
    pad = E_PAD - N_EDGES
    pad_spread = (jnp.arange(pad, dtype=jnp.int32) * 997) % N_NODES
    src3 = jnp.concatenate([src, pad_spread]) \
        .reshape(NW, NGROUPS, NBUF, CHUNK)
    dst3 = jnp.concatenate([dst, pad_spread]) \
        .reshape(NW, NGROUPS, NBUF, CHUNK)
    val3 = jnp.concatenate([edge_values, jnp.zeros((pad,), jnp.float32)]) \
        .reshape(NW, NGROUPS, NBUF, CHUNK)

    support = _support(x, W)
    partials = _aggregate(support, src3, dst3, val3)
    return _finish(partials, b.reshape(1, HIDDEN))


# speedup vs baseline: 1.0119x; 1.0119x over previous
import functools

import jax
import jax.numpy as jnp
from jax import lax
from jax.experimental import pallas as pl
from jax.experimental.pallas import tpu as pltpu
from jax.experimental.pallas import tpu_sc as plsc

N_NODES = 10000
N_EDGES = 320000
HIDDEN = 128

NUM_CORES = 2
NUM_SUBCORES = 16
NW = NUM_CORES * NUM_SUBCORES
CHUNK = 80
NBUF = 4
NGROUPS = 32
CHUNKS_PER_TILE = NBUF * NGROUPS
E_TILE = CHUNK * CHUNKS_PER_TILE
E_PAD = E_TILE * NW
ROW_CHUNK = 16
N_ROW_CHUNKS = N_NODES // ROW_CHUNK
LANES = 16


def _matmul_body(x_ref, w_ref, o_ref):
    o_ref[...] = lax.dot_general(
        x_ref[...], w_ref[...], (((1,), (0,)), ((), ())),
        preferred_element_type=jnp.float32,
        precision=lax.Precision.HIGHEST,
    )


def _support(x, W):
    return pl.pallas_call(
        _matmul_body,
        grid=(10,),
        in_specs=[
            pl.BlockSpec((N_NODES // 10, HIDDEN), lambda i: (i, 0)),
            pl.BlockSpec((HIDDEN, HIDDEN), lambda i: (0, 0)),
        ],
        out_specs=pl.BlockSpec((N_NODES // 10, HIDDEN), lambda i: (i, 0)),
        out_shape=jax.ShapeDtypeStruct((N_NODES, HIDDEN), jnp.float32),
    )(x, W)


def _finish_body(p_ref, b_ref, o_ref):
    o_ref[...] = p_ref[0] + p_ref[1] + b_ref[...]


def _finish(partials, b2d):
    return pl.pallas_call(
        _finish_body,
        grid=(10,),
        in_specs=[
            pl.BlockSpec((2, N_NODES // 10, HIDDEN), lambda i: (0, i, 0)),
            pl.BlockSpec((1, HIDDEN), lambda i: (0, 0)),
        ],
        out_specs=pl.BlockSpec((N_NODES // 10, HIDDEN), lambda i: (i, 0)),
        out_shape=jax.ShapeDtypeStruct((N_NODES, HIDDEN), jnp.float32),
    )(partials, b2d)


def _aggregate(support, src3, dst3, val3):
    mesh = plsc.VectorSubcoreMesh(core_axis_name="c", subcore_axis_name="s",
                                  num_cores=NUM_CORES, num_subcores=NUM_SUBCORES)

    @functools.partial(
        pl.kernel,
        out_type=jax.ShapeDtypeStruct((NUM_CORES, N_NODES, HIDDEN), jnp.float32),
        mesh=mesh,
        scratch_types=[
            pltpu.VMEM((2, NBUF, CHUNK), jnp.int32),
            pltpu.VMEM((2, NBUF, CHUNK), jnp.int32),
            pltpu.VMEM((2, NBUF, CHUNK), jnp.float32),
            pltpu.VMEM((NBUF, CHUNK, HIDDEN), jnp.float32),
            pltpu.VMEM_SHARED((N_NODES, HIDDEN), jnp.float32),
            [pltpu.SemaphoreType.DMA] * NBUF,
            [pltpu.SemaphoreType.DMA] * NBUF,
            [pltpu.SemaphoreType.DMA] * 2,
        ],
    )
    def agg_kernel(sup_hbm, src_hbm, dst_hbm, val_hbm, out_hbm,
                   src_v, dst_v, val_v, rows, acc, gsem, ssem, isem):
        c = lax.axis_index("c")
        s = lax.axis_index("s")
        wid = c * NUM_SUBCORES + s

        def prefetch_idx(group, slot, sem):
            pltpu.async_copy(src_hbm.at[wid, group], src_v.at[slot], sem)
            pltpu.async_copy(dst_hbm.at[wid, group], dst_v.at[slot], sem)
            pltpu.async_copy(val_hbm.at[wid, group], val_v.at[slot], sem)

        def wait_idx(slot, sem):
            pltpu.make_async_copy(src_hbm.at[wid, 0], src_v.at[slot], sem).wait()
            pltpu.make_async_copy(dst_hbm.at[wid, 0], dst_v.at[slot], sem).wait()
            pltpu.make_async_copy(val_hbm.at[wid, 0], val_v.at[slot], sem).wait()

        prefetch_idx(0, 0, isem[0])
        prefetch_idx(1, 1, isem[1])
        wait_idx(0, isem[0])

        @pl.loop(0, ROW_CHUNK)
        def _zero_rows(r):
            for h in range(0, HIDDEN, LANES):
                rows[0, r, pl.ds(h, LANES)] = jnp.zeros((LANES,), jnp.float32)

        @pl.loop(0, N_ROW_CHUNKS // NUM_SUBCORES + 1)
        def _zero_acc(k):
            cidx = s + k * NUM_SUBCORES
            @pl.when(cidx < N_ROW_CHUNKS)
            def _():
                pltpu.sync_copy(rows.at[0, pl.ds(0, ROW_CHUNK)],
                                acc.at[pl.ds(cidx * ROW_CHUNK, ROW_CHUNK)])

        plsc.subcore_barrier()

        def process_group(gg, slot, other_ready):
            for b in range(NBUF):
                pltpu.make_async_copy(sup_hbm.at[src_v.at[slot, b]],
                                      rows.at[b], gsem[b]).wait()

                @pl.loop(0, CHUNK, step=LANES)
                def _scale(g0):
                    vv = val_v[slot, b, pl.ds(g0, LANES)]
                    for e in range(LANES):
                        bc = jnp.broadcast_to(vv[e], (LANES,))
                        for h in range(0, HIDDEN, LANES):
                            rows[b, g0 + e, pl.ds(h, LANES)] = (
                                rows[b, g0 + e, pl.ds(h, LANES)] * bc)

                pltpu.async_copy(rows.at[b], acc.at[dst_v.at[slot, b]],
                                 ssem[b], add=True)

            @pl.when(gg + 1 < NGROUPS)
            def _refill():
                if other_ready is not None:
                    wait_idx(1 - slot, other_ready)
                for b in range(NBUF):
                    pltpu.make_async_copy(rows.at[b],
                                          acc.at[dst_v.at[slot, b]],
                                          ssem[b]).wait()
                    pltpu.async_copy(sup_hbm.at[src_v.at[1 - slot, b]],
                                     rows.at[b], gsem[b])

            @pl.when(gg + 2 < NGROUPS)
            def _prefetch():
                prefetch_idx(gg + 2, slot, isem[slot])

        for b in range(NBUF):
            pltpu.async_copy(sup_hbm.at[src_v.at[0, b]], rows.at[b], gsem[b])

        @pl.loop(0, NGROUPS, step=2)
        def _pair(g):
            process_group(g, 0, isem[1])
            process_group(g + 1, 1, isem[0])

        for b in range(NBUF):
            pltpu.make_async_copy(rows.at[b], acc.at[dst_v.at[1, b]],
                                  ssem[b]).wait()

        plsc.subcore_barrier()

        @pl.loop(0, N_ROW_CHUNKS // NUM_SUBCORES + 1)
        def _writeback(k):
            cidx = s + k * NUM_SUBCORES
            @pl.when(cidx < N_ROW_CHUNKS)
            def _():
                base = cidx * ROW_CHUNK
                pltpu.sync_copy(acc.at[pl.ds(base, ROW_CHUNK)],
                                out_hbm.at[c, pl.ds(base, ROW_CHUNK)])

    return agg_kernel(support, src3, dst3, val3)


def kernel(x, edge_index, edge_values, W, b):
    src = edge_index[1].astype(jnp.int32)
    dst = edge_index[0].astype(jnp.int32)
    pad = E_PAD - N_EDGES
    pad_spread = (jnp.arange(pad, dtype=jnp.int32) * 997) % N_NODES
    src3 = jnp.concatenate([src, pad_spread]) \
        .reshape(NW, NGROUPS, NBUF, CHUNK)
    dst3 = jnp.concatenate([dst, pad_spread]) \
        .reshape(NW, NGROUPS, NBUF, CHUNK)
    val3 = jnp.concatenate([edge_values, jnp.zeros((pad,), jnp.float32)]) \
        .reshape(NW, NGROUPS, NBUF, CHUNK)

    support = _support(x, W)
    partials = _aggregate(support, src3, dst3, val3)
    return _finish(partials, b.reshape(1, HIDDEN))


# speedup vs baseline: 1.0497x; 1.0373x over previous
import functools

import jax
import jax.numpy as jnp
from jax import lax
from jax.experimental import pallas as pl
from jax.experimental.pallas import tpu as pltpu
from jax.experimental.pallas import tpu_sc as plsc

N_NODES = 10000
N_EDGES = 320000
HIDDEN = 128

NUM_CORES = 2
NUM_SUBCORES = 16
NW = NUM_CORES * NUM_SUBCORES
CHUNK = 80
NBUF = 4
GROUP_E = NBUF * CHUNK
NGROUPS = 32
E_TILE = GROUP_E * NGROUPS
TAIL_GROUPS = 8
ROW_CHUNK = 16
N_ROW_CHUNKS = N_NODES // ROW_CHUNK
LANES = 16


def _matmul_body(x_ref, w_ref, o_ref):
    o_ref[...] = lax.dot_general(
        x_ref[...], w_ref[...], (((1,), (0,)), ((), ())),
        preferred_element_type=jnp.float32,
        precision=lax.Precision.HIGHEST,
    )


def _support(x, W):
    return pl.pallas_call(
        _matmul_body,
        grid=(10,),
        in_specs=[
            pl.BlockSpec((N_NODES // 10, HIDDEN), lambda i: (i, 0)),
            pl.BlockSpec((HIDDEN, HIDDEN), lambda i: (0, 0)),
        ],
        out_specs=pl.BlockSpec((N_NODES // 10, HIDDEN), lambda i: (i, 0)),
        out_shape=jax.ShapeDtypeStruct((N_NODES, HIDDEN), jnp.float32),
    )(x, W)


def _finish_body(p_ref, b_ref, o_ref):
    o_ref[...] = p_ref[0] + p_ref[1] + b_ref[...]


def _finish(partials, b2d):
    return pl.pallas_call(
        _finish_body,
        grid=(10,),
        in_specs=[
            pl.BlockSpec((2, N_NODES // 10, HIDDEN), lambda i: (0, i, 0)),
            pl.BlockSpec((1, HIDDEN), lambda i: (0, 0)),
        ],
        out_specs=pl.BlockSpec((N_NODES // 10, HIDDEN), lambda i: (i, 0)),
        out_shape=jax.ShapeDtypeStruct((N_NODES, HIDDEN), jnp.float32),
    )(partials, b2d)


def _aggregate(support, src, dst, val):
    mesh = plsc.VectorSubcoreMesh(core_axis_name="c", subcore_axis_name="s",
                                  num_cores=NUM_CORES, num_subcores=NUM_SUBCORES)

    @functools.partial(
        pl.kernel,
        out_type=jax.ShapeDtypeStruct((NUM_CORES, N_NODES, HIDDEN), jnp.float32),
        mesh=mesh,
        scratch_types=[
            [pltpu.VMEM((GROUP_E,), jnp.int32)] * 2,
            [pltpu.VMEM((GROUP_E,), jnp.int32)] * 2,
            pltpu.VMEM((2, NBUF, CHUNK), jnp.int32),
            [pltpu.VMEM((GROUP_E,), jnp.float32)] * 2,
            pltpu.VMEM((NBUF, CHUNK, HIDDEN), jnp.float32),
            pltpu.VMEM_SHARED((N_NODES, HIDDEN), jnp.float32),
            [pltpu.SemaphoreType.DMA] * NBUF,
            [pltpu.SemaphoreType.DMA] * NBUF,
            [pltpu.SemaphoreType.DMA] * 2,
        ],
    )
    def agg_kernel(sup_hbm, src_hbm, dst_hbm, val_hbm, out_hbm,
                   src_v, dst_s, dst_v, val_v, rows, acc, gsem, ssem, isem):
        c = lax.axis_index("c")
        s = lax.axis_index("s")
        wid = c * NUM_SUBCORES + s
        ng = jnp.where(wid == NW - 1, TAIL_GROUPS, NGROUPS)

        def prefetch_idx(group, slot, sem):
            base = wid * E_TILE + group * GROUP_E
            pltpu.async_copy(src_hbm.at[pl.ds(base, GROUP_E)],
                             src_v[slot], sem)
            pltpu.async_copy(dst_hbm.at[pl.ds(base, GROUP_E)],
                             dst_s[slot], sem)
            pltpu.async_copy(val_hbm.at[pl.ds(base, GROUP_E)],
                             val_v[slot], sem)

        def wait_idx(slot, sem):
            pltpu.make_async_copy(src_hbm.at[pl.ds(0, GROUP_E)],
                                  src_v[slot], sem).wait()
            pltpu.make_async_copy(dst_hbm.at[pl.ds(0, GROUP_E)],
                                  dst_s[slot], sem).wait()
            pltpu.make_async_copy(val_hbm.at[pl.ds(0, GROUP_E)],
                                  val_v[slot], sem).wait()
            for b in range(NBUF):
                for k in range(0, CHUNK, LANES):
                    dst_v[slot, b, pl.ds(k, LANES)] = (
                        dst_s[slot][pl.ds(b * CHUNK + k, LANES)])

        prefetch_idx(0, 0, isem[0])
        prefetch_idx(1, 1, isem[1])
        wait_idx(0, isem[0])

        @pl.loop(0, ROW_CHUNK)
        def _zero_rows(r):
            for h in range(0, HIDDEN, LANES):
                rows[0, r, pl.ds(h, LANES)] = jnp.zeros((LANES,), jnp.float32)

        @pl.loop(0, N_ROW_CHUNKS // NUM_SUBCORES + 1)
        def _zero_acc(k):
            cidx = s + k * NUM_SUBCORES
            @pl.when(cidx < N_ROW_CHUNKS)
            def _():
                pltpu.sync_copy(rows.at[0, pl.ds(0, ROW_CHUNK)],
                                acc.at[pl.ds(cidx * ROW_CHUNK, ROW_CHUNK)])

        plsc.subcore_barrier()

        def process_group(gg, slot, other_ready):
            for b in range(NBUF):
                pltpu.make_async_copy(
                    sup_hbm.at[src_v[slot].at[pl.ds(b * CHUNK, CHUNK)]],
                    rows.at[b], gsem[b]).wait()

                @pl.loop(0, CHUNK, step=LANES)
                def _scale(g0):
                    vv = val_v[slot][pl.ds(b * CHUNK + g0, LANES)]
                    for e in range(LANES):
                        bc = jnp.broadcast_to(vv[e], (LANES,))
                        for h in range(0, HIDDEN, LANES):
                            rows[b, g0 + e, pl.ds(h, LANES)] = (
                                rows[b, g0 + e, pl.ds(h, LANES)] * bc)

                pltpu.async_copy(rows.at[b], acc.at[dst_v.at[slot, b]],
                                 ssem[b], add=True)

            @pl.when(gg + 1 < ng)
            def _refill():
                if other_ready is not None:
                    wait_idx(1 - slot, other_ready)
                for b in range(NBUF):
                    pltpu.make_async_copy(rows.at[b],
                                          acc.at[dst_v.at[slot, b]],
                                          ssem[b]).wait()
                    pltpu.async_copy(
                        sup_hbm.at[src_v[1 - slot].at[pl.ds(b * CHUNK, CHUNK)]],
                        rows.at[b], gsem[b])

            @pl.when(gg + 2 < ng)
            def _prefetch():
                prefetch_idx(gg + 2, slot, isem[slot])

        for b in range(NBUF):
            pltpu.async_copy(sup_hbm.at[src_v[0].at[pl.ds(b * CHUNK, CHUNK)]],
                             rows.at[b], gsem[b])

        @pl.loop(0, ng, step=2)
        def _pair(g):
            process_group(g, 0, isem[1])
            process_group(g + 1, 1, isem[0])

        for b in range(NBUF):
            pltpu.make_async_copy(rows.at[b], acc.at[dst_v.at[1, b]],
                                  ssem[b]).wait()

        plsc.subcore_barrier()

        @pl.loop(0, N_ROW_CHUNKS // NUM_SUBCORES + 1)
        def _writeback(k):
            cidx = s + k * NUM_SUBCORES
            @pl.when(cidx < N_ROW_CHUNKS)
            def _():
                base = cidx * ROW_CHUNK
                pltpu.sync_copy(acc.at[pl.ds(base, ROW_CHUNK)],
                                out_hbm.at[c, pl.ds(base, ROW_CHUNK)])

    return agg_kernel(support, src, dst, val)


def kernel(x, edge_index, edge_values, W, b):
    src = edge_index[1].astype(jnp.int32)
    dst = edge_index[0].astype(jnp.int32)
    support = _support(x, W)
    partials = _aggregate(support, src, dst, edge_values)
    return _finish(partials, b.reshape(1, HIDDEN))
